# baseline (speedup 1.0000x reference)
import functools

import jax
import jax.numpy as jnp
from jax import lax
from jax.experimental import pallas as pl
from jax.experimental.pallas import tpu as pltpu
from jax.experimental.pallas import tpu_sc as plsc

NC, NS, L = 2, 16, 16
NW = NC * NS
N = 10000
NPAD = 10240
SLICE = NPAD // NS
E = 320000
EPT = 10240
EP = NW * EPT
D = 128
CH = 64
NB = 40
NBLK = EPT // (NB * CH)
NBH = 2 * NBLK
NBUF = 4

_mesh = plsc.VectorSubcoreMesh(core_axis_name="c", subcore_axis_name="s")


@functools.partial(
    pl.kernel,
    out_type=(
        jax.ShapeDtypeStruct((NC, NPAD, D), jnp.float32),
        jax.ShapeDtypeStruct((NC * NPAD, D), jnp.float32),
        jax.ShapeDtypeStruct((NC, NS, SLICE), jnp.float32),
    ),
    mesh=_mesh,
    scratch_types=[
        pltpu.VMEM((NB, CH), jnp.int32),
        pltpu.VMEM((NB, CH), jnp.int32),
        pltpu.VMEM((CH, D), jnp.float32),
        pltpu.VMEM((CH, D), jnp.float32),
        pltpu.VMEM((CH, D), jnp.float32),
        pltpu.VMEM((CH, D), jnp.float32),
        pltpu.VMEM((SLICE + L,), jnp.float32),
        pltpu.VMEM_SHARED((NPAD,), jnp.float32),
        pltpu.VMEM_SHARED((NPAD, D), jnp.float32),
        pltpu.SemaphoreType.DMA,
        pltpu.SemaphoreType.DMA,
        pltpu.SemaphoreType.DMA,
        pltpu.SemaphoreType.DMA,
        pltpu.SemaphoreType.DMA,
        pltpu.SemaphoreType.DMA,
        pltpu.SemaphoreType.DMA,
        pltpu.SemaphoreType.DMA,
    ],
)
def _gcn_sc_kernel(dsth_hbm, src_hbm, dst_hbm, xp_hbm, initv_hbm,
                   s_out, xs_out, dis_out,
                   b0, b1, r0, r1, r2, r3, deg_l, deg_sp, agg_sp,
                   g0, g1, g2, g3, t0, t1, t2, t3):
    c = lax.axis_index("c")
    s = lax.axis_index("s")
    wid = c * NS + s
    rows = (r0, r1, r2, r3)
    gsem = (g0, g1, g2, g3)
    ssem = (t0, t1, t2, t3)

    pltpu.sync_copy(initv_hbm, deg_l.at[pl.ds(0, CH)])

    def zr(r, _):
        for j in range(D // L):
            r1[r, pl.ds(j * L, L)] = jnp.zeros((L,), jnp.float32)
        return 0
    lax.fori_loop(0, CH, zr, 0)
    for k in range(SLICE // CH):
        pltpu.sync_copy(r1, agg_sp.at[pl.ds(s * SLICE + k * CH, CH)])
        pltpu.sync_copy(deg_l.at[pl.ds(0, CH)],
                        deg_sp.at[pl.ds(s * SLICE + k * CH, CH)])

    def fl(i, _):
        deg_l[pl.ds(i * L, L)] = jnp.ones((L,), jnp.float32)
        return 0
    lax.fori_loop(0, CH // L, fl, 0)
    plsc.subcore_barrier()
    ones_v = deg_l.at[pl.ds(0, CH)]
    bufs = (b0, b1)

    def fire(buf, sem):
        def one(i, _):
            pltpu.async_copy(ones_v, deg_sp.at[buf.at[i]], sem, add=True)
            return 0
        lax.fori_loop(0, NB, one, 0)

    def drain(buf, sem):
        def one(i, _):
            pltpu.make_async_copy(ones_v, deg_sp.at[buf.at[0]], sem).wait()
            return 0
        lax.fori_loop(0, NB, one, 0)

    for blk in range(NBH):
        p = blk % 2
        if blk >= 2:
            drain(bufs[p], gsem[p])
        pltpu.sync_copy(dsth_hbm.at[s, blk], bufs[p])
        fire(bufs[p], gsem[p])
    for blk in range(NBH - 2, NBH):
        p = blk % 2
        drain(bufs[p], gsem[p])
    plsc.subcore_barrier()

    pltpu.sync_copy(deg_sp.at[pl.ds(s * SLICE, SLICE)],
                    deg_l.at[pl.ds(0, SLICE)])

    def newton(m, _):
        d = deg_l[pl.ds(m * L, L)]
        bits = lax.bitcast_convert_type(d, jnp.int32)
        y = lax.bitcast_convert_type(
            0x5F3759DF - lax.shift_right_logical(bits, 1), jnp.float32)
        hd = 0.5 * d
        for _ in range(3):
            y = y * (1.5 - hd * y * y)
        deg_l[pl.ds(m * L, L)] = y
        return 0
    lax.fori_loop(0, SLICE // L, newton, 0)
    pltpu.sync_copy(deg_l.at[pl.ds(0, SLICE)], dis_out.at[c, s])

    NXC = SLICE // CH

    def xload(k, buf, sem):
        pltpu.async_copy(xp_hbm.at[pl.ds(s * SLICE + k * CH, CH)], buf, sem)

    def xwait(buf, sem):
        pltpu.make_async_copy(xp_hbm.at[pl.ds(0, CH)], buf, sem).wait()

    xload(0, r1, gsem[0])
    for k in range(NXC):
        p = k % 2
        buf = rows[1 + p]
        if k + 1 < NXC:
            xload(k + 1, rows[1 + (1 - p)], gsem[1 - p])
        xwait(buf, gsem[p])

        def scale(r, _):
            v = deg_l[pl.ds(k * CH + r, L)][0]
            for j in range(D // L):
                buf[r, pl.ds(j * L, L)] = buf[r, pl.ds(j * L, L)] * v
            return 0
        lax.fori_loop(0, CH, scale, 0)
        if k >= 2:
            pltpu.make_async_copy(
                buf, xs_out.at[pl.ds(0, CH)], ssem[p]).wait()
        pltpu.async_copy(
            buf, xs_out.at[pl.ds(c * NPAD + s * SLICE + k * CH, CH)],
            ssem[p])
    for p in range(2):
        pltpu.make_async_copy(r1, xs_out.at[pl.ds(0, CH)], ssem[p]).wait()
    plsc.subcore_barrier()

    def gstart(i, k):
        pltpu.async_copy(xs_out.at[b0.at[i]], rows[k], gsem[k])

    def gwait(k):
        pltpu.make_async_copy(xs_out.at[b0.at[0]], rows[k], gsem[k]).wait()

    def sstart(i, k):
        pltpu.async_copy(rows[k], agg_sp.at[b1.at[i]], ssem[k], add=True)

    def swait(k):
        pltpu.make_async_copy(rows[k], agg_sp.at[b1.at[0]], ssem[k]).wait()

    for blk in range(NBLK):
        pltpu.sync_copy(src_hbm.at[wid, blk], b0)
        pltpu.sync_copy(dst_hbm.at[wid, blk], b1)
        for k in range(NBUF):
            gstart(k, k)

        def quad(j, _):
            for k in range(NBUF):
                i = NBUF * j + k
                gwait(k)
                sstart(i, k)
                swait(k)
                gstart(i + NBUF, k)
            return 0
        lax.fori_loop(0, (NB - NBUF) // NBUF, quad, 0)

        for k in range(NBUF):
            gwait(k)
            sstart(NB - NBUF + k, k)
        for k in range(NBUF):
            swait(k)

    plsc.subcore_barrier()
    pltpu.sync_copy(agg_sp.at[pl.ds(s * SLICE, SLICE)],
                    s_out.at[c, pl.ds(s * SLICE, SLICE)])


def _combine_body(s_ref, xs_ref, dis_ref, wt_ref, out_ref):
    agg = s_ref[0] + s_ref[1] + xs_ref[...]
    a = agg * dis_ref[0]
    out_ref[...] = jnp.dot(a, wt_ref[...], preferred_element_type=jnp.float32)


def _combine(s2, xs, dis3, wt):
    rb = 400
    grid = N // rb
    return pl.pallas_call(
        _combine_body,
        grid=(grid,),
        in_specs=[
            pl.BlockSpec((NC, rb, D), lambda i: (0, i, 0)),
            pl.BlockSpec((rb, D), lambda i: (i, 0)),
            pl.BlockSpec((1, rb, 1), lambda i: (0, i, 0)),
            pl.BlockSpec((D, D), lambda i: (0, 0)),
        ],
        out_specs=pl.BlockSpec((rb, D), lambda i: (i, 0)),
        out_shape=jax.ShapeDtypeStruct((N, D), jnp.float32),
    )(s2, xs, dis3, wt)


def kernel(x, edge_index, num_nodes, W):
    pad = jnp.arange(EP - E, dtype=jnp.int32)
    srcp = jnp.concatenate([edge_index[0].astype(jnp.int32), pad % N])
    dstp = jnp.concatenate(
        [edge_index[1].astype(jnp.int32), N + pad % (NPAD - N)])
    offs = (jnp.arange(NW, dtype=jnp.int32) // NS) * NPAD
    src4 = (srcp.reshape(NW, EPT) + offs[:, None]).reshape(NW, NBLK, NB, CH)
    dst4 = dstp.reshape(NW, NBLK, NB, CH)
    dsth = dstp.reshape(NS, NBH, NB, CH)
    xp = jnp.pad(x, ((0, NPAD - N), (0, 0)))
    resid = jnp.asarray(num_nodes, jnp.float32) - x.shape[0]
    initv = jnp.full((CH,), 1.0, jnp.float32) + resid

    s_p, xs_c, dis = _gcn_sc_kernel(dsth, src4, dst4, xp, initv)
    dis3 = dis.reshape(NC, NPAD, 1)
    return _combine(s_p, xs_c, dis3, W.T)


# speedup vs baseline: 1.0797x; 1.0797x over previous
import functools

import jax
import jax.numpy as jnp
from jax import lax
from jax.experimental import pallas as pl
from jax.experimental.pallas import tpu as pltpu
from jax.experimental.pallas import tpu_sc as plsc

NC, NS, L = 2, 16, 16
NW = NC * NS
N = 10000
NPAD = 10240
SLICE = NPAD // NS
E = 320000
EPT = 10240
EP = NW * EPT
D = 128
CH = 64
NB = 40
NBLK = EPT // (NB * CH)
NBUF = 4

_mesh = plsc.VectorSubcoreMesh(core_axis_name="c", subcore_axis_name="s")


@functools.partial(
    pl.kernel,
    out_type=jax.ShapeDtypeStruct((NC, NPAD), jnp.float32),
    mesh=_mesh,
    scratch_types=[
        pltpu.VMEM((NB, CH), jnp.int32),
        pltpu.VMEM((NB, CH), jnp.int32),
        pltpu.VMEM((CH,), jnp.float32),
        pltpu.VMEM_SHARED((NPAD,), jnp.float32),
        pltpu.SemaphoreType.DMA,
        pltpu.SemaphoreType.DMA,
    ],
)
def _degree_kernel(dst_hbm, deg_out, dst_b0, dst_b1, ones_v, deg_sp,
                   sem0, sem1):
    c = lax.axis_index("c")
    s = lax.axis_index("s")
    wid = c * NS + s

    def fill(i, val):
        ones_v[pl.ds(i * L, L)] = jnp.full((L,), val, jnp.float32)
        return val
    lax.fori_loop(0, CH // L, fill, 0.0)
    for k in range(SLICE // CH):
        pltpu.sync_copy(ones_v, deg_sp.at[pl.ds(s * SLICE + k * CH, CH)])
    plsc.subcore_barrier()
    lax.fori_loop(0, CH // L, fill, 1.0)

    bufs = (dst_b0, dst_b1)
    sems = (sem0, sem1)

    def fire(buf, sem):
        def one(i, _):
            pltpu.async_copy(ones_v, deg_sp.at[buf.at[i]], sem, add=True)
            return 0
        lax.fori_loop(0, NB, one, 0)

    def drain(buf, sem):
        def one(i, _):
            pltpu.make_async_copy(ones_v, deg_sp.at[buf.at[0]], sem).wait()
            return 0
        lax.fori_loop(0, NB, one, 0)

    for blk in range(NBLK):
        p = blk % 2
        if blk >= 2:
            drain(bufs[p], sems[p])
        pltpu.sync_copy(dst_hbm.at[wid, blk], bufs[p])
        fire(bufs[p], sems[p])
    for blk in range(NBLK - 2, NBLK):
        p = blk % 2
        drain(bufs[p], sems[p])

    plsc.subcore_barrier()
    pltpu.sync_copy(deg_sp.at[pl.ds(s * SLICE, SLICE)],
                    deg_out.at[c, pl.ds(s * SLICE, SLICE)])


@functools.partial(
    pl.kernel,
    out_type=jax.ShapeDtypeStruct((NC, NPAD, D), jnp.float32),
    mesh=_mesh,
    scratch_types=[
        pltpu.VMEM((NB, CH), jnp.int32),
        pltpu.VMEM((NB, CH), jnp.int32),
        pltpu.VMEM((CH, D), jnp.float32),
        pltpu.VMEM((CH, D), jnp.float32),
        pltpu.VMEM((CH, D), jnp.float32),
        pltpu.VMEM((CH, D), jnp.float32),
        pltpu.VMEM_SHARED((NPAD, D), jnp.float32),
        pltpu.SemaphoreType.DMA,
        pltpu.SemaphoreType.DMA,
        pltpu.SemaphoreType.DMA,
        pltpu.SemaphoreType.DMA,
        pltpu.SemaphoreType.DMA,
        pltpu.SemaphoreType.DMA,
        pltpu.SemaphoreType.DMA,
        pltpu.SemaphoreType.DMA,
    ],
)
def _scatter_kernel(src_hbm, dst_hbm, xs_hbm, s_out,
                    src_blk, dst_blk, r0, r1, r2, r3, agg_sp,
                    g0, g1, g2, g3, s0, s1, s2, s3):
    c = lax.axis_index("c")
    s = lax.axis_index("s")
    wid = c * NS + s
    rows = (r0, r1, r2, r3)
    gsem = (g0, g1, g2, g3)
    ssem = (s0, s1, s2, s3)

    def zr(r, _):
        for j in range(D // L):
            r0[r, pl.ds(j * L, L)] = jnp.zeros((L,), jnp.float32)
        return 0
    lax.fori_loop(0, CH, zr, 0)
    for k in range(SLICE // CH):
        pltpu.sync_copy(r0, agg_sp.at[pl.ds(s * SLICE + k * CH, CH)])
    plsc.subcore_barrier()

    def gstart(i, k):
        pltpu.async_copy(xs_hbm.at[src_blk.at[i]], rows[k], gsem[k])

    def gwait(k):
        pltpu.make_async_copy(xs_hbm.at[src_blk.at[0]], rows[k],
                              gsem[k]).wait()

    def sstart(i, k):
        pltpu.async_copy(rows[k], agg_sp.at[dst_blk.at[i]], ssem[k],
                         add=True)

    def swait(k):
        pltpu.make_async_copy(rows[k], agg_sp.at[dst_blk.at[0]],
                              ssem[k]).wait()

    for blk in range(NBLK):
        pltpu.sync_copy(src_hbm.at[wid, blk], src_blk)
        pltpu.sync_copy(dst_hbm.at[wid, blk], dst_blk)
        gstart(0, 0)
        gstart(1, 1)
        gstart(2, 2)
        gwait(0)
        sstart(0, 0)
        gstart(3, 3)
        gwait(1)
        sstart(1, 1)

        def quad(j, _):
            for k in range(NBUF):
                i = NBUF * j + 2 + k
                b = (2 + k) % NBUF
                b2 = k % NBUF
                swait(b2)
                gstart(i + 2, b2)
                gwait(b)
                sstart(i, b)
            return 0
        lax.fori_loop(0, (NB - NBUF) // NBUF, quad, 0)

        swait(0)
        gwait(2)
        sstart(NB - 2, 2)
        swait(1)
        gwait(3)
        sstart(NB - 1, 3)
        swait(2)
        swait(3)

    plsc.subcore_barrier()
    pltpu.sync_copy(agg_sp.at[pl.ds(s * SLICE, SLICE)],
                    s_out.at[c, pl.ds(s * SLICE, SLICE)])


def _prescale_body(deg_ref, x_ref, adj_ref, xs_ref, dis_ref):
    deg = deg_ref[0, :N] + deg_ref[1, :N] + 1.0 + adj_ref[0, 0]
    dis = lax.rsqrt(deg)
    dis_ref[...] = dis
    xs_ref[...] = x_ref[...] * dis


def _prescale(deg2, x, adj):
    return pl.pallas_call(
        _prescale_body,
        out_shape=[
            jax.ShapeDtypeStruct((N, D), jnp.float32),
            jax.ShapeDtypeStruct((N, 1), jnp.float32),
        ],
    )(deg2, x, adj)


def _combine_body(s_ref, xs_ref, dis_ref, wt_ref, out_ref):
    agg = s_ref[0] + s_ref[1] + xs_ref[...]
    a = agg * dis_ref[...]
    out_ref[...] = jnp.dot(a, wt_ref[...], preferred_element_type=jnp.float32)


def _combine(s2, xs, dis, wt):
    rb = 400
    grid = N // rb
    return pl.pallas_call(
        _combine_body,
        grid=(grid,),
        in_specs=[
            pl.BlockSpec((NC, rb, D), lambda i: (0, i, 0)),
            pl.BlockSpec((rb, D), lambda i: (i, 0)),
            pl.BlockSpec((rb, 1), lambda i: (i, 0)),
            pl.BlockSpec((D, D), lambda i: (0, 0)),
        ],
        out_specs=pl.BlockSpec((rb, D), lambda i: (i, 0)),
        out_shape=jax.ShapeDtypeStruct((N, D), jnp.float32),
    )(s2, xs, dis, wt)


def kernel(x, edge_index, num_nodes, W):
    pad = jnp.arange(EP - E, dtype=jnp.int32)
    srcp = jnp.concatenate([edge_index[0].astype(jnp.int32), pad % N])
    dstp = jnp.concatenate(
        [edge_index[1].astype(jnp.int32), N + pad % (NPAD - N)])
    src4 = srcp.reshape(NW, NBLK, NB, CH)
    dst4 = dstp.reshape(NW, NBLK, NB, CH)
    adj = (jnp.asarray(num_nodes, jnp.float32) - x.shape[0]).reshape(1, 1)

    deg_p = _degree_kernel(dst4)
    xs, dis = _prescale(deg_p.reshape(NC, NPAD, 1), x, adj)
    s_p = _scatter_kernel(src4, dst4, xs)
    return _combine(s_p, xs, dis, W.T)


# speedup vs baseline: 1.2056x; 1.1165x over previous
import functools

import jax
import jax.numpy as jnp
from jax import lax
from jax.experimental import pallas as pl
from jax.experimental.pallas import tpu as pltpu
from jax.experimental.pallas import tpu_sc as plsc

NC, NS, L = 2, 16, 16
NW = NC * NS
N = 10000
NPAD = 10240
SLICE = NPAD // NS
E = 320000
EPT = 10240
EP = NW * EPT
D = 128
CH = 64
NB = 40
NBLK = EPT // (NB * CH)
NBUF = 4

_mesh = plsc.VectorSubcoreMesh(core_axis_name="c", subcore_axis_name="s")


@functools.partial(
    pl.kernel,
    out_type=jax.ShapeDtypeStruct((NC, NPAD), jnp.float32),
    mesh=_mesh,
    scratch_types=[
        pltpu.VMEM((NB, CH), jnp.int32),
        pltpu.VMEM((NB, CH), jnp.int32),
        pltpu.VMEM((CH,), jnp.float32),
        pltpu.VMEM_SHARED((NPAD,), jnp.float32),
        pltpu.SemaphoreType.DMA,
        pltpu.SemaphoreType.DMA,
    ],
)
def _degree_kernel(dst_hbm, deg_out, dst_b0, dst_b1, ones_v, deg_sp,
                   sem0, sem1):
    c = lax.axis_index("c")
    s = lax.axis_index("s")
    wid = c * NS + s

    def fill(i, val):
        ones_v[pl.ds(i * L, L)] = jnp.full((L,), val, jnp.float32)
        return val
    lax.fori_loop(0, CH // L, fill, 0.0)
    for k in range(SLICE // CH):
        pltpu.sync_copy(ones_v, deg_sp.at[pl.ds(s * SLICE + k * CH, CH)])
    plsc.subcore_barrier()
    lax.fori_loop(0, CH // L, fill, 1.0)

    bufs = (dst_b0, dst_b1)
    sems = (sem0, sem1)

    def fire(buf, sem):
        def one(i, _):
            pltpu.async_copy(ones_v, deg_sp.at[buf.at[i]], sem, add=True)
            return 0
        lax.fori_loop(0, NB, one, 0)

    def drain(buf, sem):
        def one(i, _):
            pltpu.make_async_copy(ones_v, deg_sp.at[buf.at[0]], sem).wait()
            return 0
        lax.fori_loop(0, NB, one, 0)

    for blk in range(NBLK):
        p = blk % 2
        if blk >= 2:
            drain(bufs[p], sems[p])
        pltpu.sync_copy(dst_hbm.at[wid, blk], bufs[p])
        fire(bufs[p], sems[p])
    for blk in range(NBLK - 2, NBLK):
        p = blk % 2
        drain(bufs[p], sems[p])

    plsc.subcore_barrier()
    pltpu.sync_copy(deg_sp.at[pl.ds(s * SLICE, SLICE)],
                    deg_out.at[c, pl.ds(s * SLICE, SLICE)])


@functools.partial(
    pl.kernel,
    out_type=jax.ShapeDtypeStruct((NC, NPAD, D), jnp.float32),
    mesh=_mesh,
    scratch_types=[
        pltpu.VMEM((NB, CH), jnp.int32),
        pltpu.VMEM((NB, CH), jnp.int32),
        pltpu.VMEM((CH, D), jnp.float32),
        pltpu.VMEM((CH, D), jnp.float32),
        pltpu.VMEM((CH, D), jnp.float32),
        pltpu.VMEM((CH, D), jnp.float32),
        pltpu.VMEM_SHARED((NPAD, D), jnp.float32),
        pltpu.SemaphoreType.DMA,
        pltpu.SemaphoreType.DMA,
        pltpu.SemaphoreType.DMA,
        pltpu.SemaphoreType.DMA,
        pltpu.SemaphoreType.DMA,
        pltpu.SemaphoreType.DMA,
        pltpu.SemaphoreType.DMA,
        pltpu.SemaphoreType.DMA,
    ],
)
def _scatter_kernel(src_hbm, dst_hbm, xs_hbm, s_out,
                    src_blk, dst_blk, r0, r1, r2, r3, agg_sp,
                    g0, g1, g2, g3, s0, s1, s2, s3):
    c = lax.axis_index("c")
    s = lax.axis_index("s")
    wid = c * NS + s
    rows = (r0, r1, r2, r3)
    gsem = (g0, g1, g2, g3)
    ssem = (s0, s1, s2, s3)

    def zr(r, _):
        for j in range(D // L):
            r0[r, pl.ds(j * L, L)] = jnp.zeros((L,), jnp.float32)
        return 0
    lax.fori_loop(0, CH, zr, 0)
    for k in range(SLICE // CH):
        pltpu.sync_copy(r0, agg_sp.at[pl.ds(s * SLICE + k * CH, CH)])
    plsc.subcore_barrier()

    def gstart(i, k):
        pltpu.async_copy(xs_hbm.at[src_blk.at[i]], rows[k], gsem[k])

    def gwait(k):
        pltpu.make_async_copy(xs_hbm.at[src_blk.at[0]], rows[k],
                              gsem[k]).wait()

    def sstart(i, k):
        pltpu.async_copy(rows[k], agg_sp.at[dst_blk.at[i]], ssem[k],
                         add=True)

    def swait(k):
        pltpu.make_async_copy(rows[k], agg_sp.at[dst_blk.at[0]],
                              ssem[k]).wait()

    for blk in range(NBLK):
        pltpu.sync_copy(src_hbm.at[wid, blk], src_blk)
        pltpu.sync_copy(dst_hbm.at[wid, blk], dst_blk)
        for k in range(NBUF):
            gstart(k, k)

        def quad(j, _):
            for k in range(NBUF):
                i = NBUF * j + k
                gwait(k)
                sstart(i, k)
                swait(k)
                gstart(i + NBUF, k)
            return 0
        lax.fori_loop(0, (NB - NBUF) // NBUF, quad, 0)

        for k in range(NBUF):
            gwait(k)
            sstart(NB - NBUF + k, k)
        for k in range(NBUF):
            swait(k)

    plsc.subcore_barrier()
    pltpu.sync_copy(agg_sp.at[pl.ds(s * SLICE, SLICE)],
                    s_out.at[c, pl.ds(s * SLICE, SLICE)])


def _prescale_body(deg_ref, x_ref, adj_ref, xs_ref, dis_ref):
    deg = deg_ref[0, :N] + deg_ref[1, :N] + 1.0 + adj_ref[0, 0]
    dis = lax.rsqrt(deg)
    dis_ref[...] = dis
    xs_ref[...] = x_ref[...] * dis


def _prescale(deg2, x, adj):
    return pl.pallas_call(
        _prescale_body,
        out_shape=[
            jax.ShapeDtypeStruct((N, D), jnp.float32),
            jax.ShapeDtypeStruct((N, 1), jnp.float32),
        ],
    )(deg2, x, adj)


def _combine_body(s_ref, xs_ref, dis_ref, wt_ref, out_ref):
    agg = s_ref[0] + s_ref[1] + xs_ref[...]
    a = agg * dis_ref[...]
    out_ref[...] = jnp.dot(a, wt_ref[...], preferred_element_type=jnp.float32)


def _combine(s2, xs, dis, wt):
    rb = 2000
    grid = N // rb
    return pl.pallas_call(
        _combine_body,
        grid=(grid,),
        in_specs=[
            pl.BlockSpec((NC, rb, D), lambda i: (0, i, 0)),
            pl.BlockSpec((rb, D), lambda i: (i, 0)),
            pl.BlockSpec((rb, 1), lambda i: (i, 0)),
            pl.BlockSpec((D, D), lambda i: (0, 0)),
        ],
        out_specs=pl.BlockSpec((rb, D), lambda i: (i, 0)),
        out_shape=jax.ShapeDtypeStruct((N, D), jnp.float32),
    )(s2, xs, dis, wt)


def kernel(x, edge_index, num_nodes, W):
    pad = jnp.arange(EP - E, dtype=jnp.int32)
    srcp = jnp.concatenate([edge_index[0].astype(jnp.int32), pad % N])
    dstp = jnp.concatenate(
        [edge_index[1].astype(jnp.int32), N + pad % (NPAD - N)])
    src4 = srcp.reshape(NW, NBLK, NB, CH)
    dst4 = dstp.reshape(NW, NBLK, NB, CH)
    adj = (jnp.asarray(num_nodes, jnp.float32) - x.shape[0]).reshape(1, 1)

    deg_p = _degree_kernel(dst4)
    xs, dis = _prescale(deg_p.reshape(NC, NPAD, 1), x, adj)
    s_p = _scatter_kernel(src4, dst4, xs)
    return _combine(s_p, xs, dis, W.T)


# speedup vs baseline: 1.2093x; 1.0031x over previous
import functools

import jax
import jax.numpy as jnp
from jax import lax
from jax.experimental import pallas as pl
from jax.experimental.pallas import tpu as pltpu
from jax.experimental.pallas import tpu_sc as plsc

NC, NS, L = 2, 16, 16
NW = NC * NS
N = 10000
NPAD = 10240
SLICE = NPAD // NS
E = 320000
EPT = 10240
EP = NW * EPT
D = 128
CH = 64
NB = 40
NBLK = EPT // (NB * CH)
NBUF = 4

_mesh = plsc.VectorSubcoreMesh(core_axis_name="c", subcore_axis_name="s")


@functools.partial(
    pl.kernel,
    out_type=jax.ShapeDtypeStruct((NC, NPAD), jnp.float32),
    mesh=_mesh,
    scratch_types=[
        pltpu.VMEM((NB, CH), jnp.int32),
        pltpu.VMEM((NB, CH), jnp.int32),
        pltpu.VMEM((CH,), jnp.float32),
        pltpu.VMEM_SHARED((NPAD,), jnp.float32),
        pltpu.SemaphoreType.DMA,
        pltpu.SemaphoreType.DMA,
    ],
)
def _degree_kernel(dst_hbm, deg_out, dst_b0, dst_b1, ones_v, deg_sp,
                   sem0, sem1):
    c = lax.axis_index("c")
    s = lax.axis_index("s")
    wid = c * NS + s

    def fill(i, val):
        ones_v[pl.ds(i * L, L)] = jnp.full((L,), val, jnp.float32)
        return val
    lax.fori_loop(0, CH // L, fill, 0.0)
    for k in range(SLICE // CH):
        pltpu.sync_copy(ones_v, deg_sp.at[pl.ds(s * SLICE + k * CH, CH)])
    plsc.subcore_barrier()
    lax.fori_loop(0, CH // L, fill, 1.0)

    bufs = (dst_b0, dst_b1)
    sems = (sem0, sem1)

    def fire(buf, sem):
        def one(i, _):
            pltpu.async_copy(ones_v, deg_sp.at[buf.at[i]], sem, add=True)
            return 0
        lax.fori_loop(0, NB, one, 0)

    def drain(buf, sem):
        def one(i, _):
            pltpu.make_async_copy(ones_v, deg_sp.at[buf.at[0]], sem).wait()
            return 0
        lax.fori_loop(0, NB, one, 0)

    for blk in range(NBLK):
        p = blk % 2
        if blk >= 2:
            drain(bufs[p], sems[p])
        pltpu.sync_copy(dst_hbm.at[wid, blk], bufs[p])
        fire(bufs[p], sems[p])
    for blk in range(NBLK - 2, NBLK):
        p = blk % 2
        drain(bufs[p], sems[p])

    plsc.subcore_barrier()
    pltpu.sync_copy(deg_sp.at[pl.ds(s * SLICE, SLICE)],
                    deg_out.at[c, pl.ds(s * SLICE, SLICE)])


@functools.partial(
    pl.kernel,
    out_type=jax.ShapeDtypeStruct((NC, NPAD, D), jnp.float32),
    mesh=_mesh,
    scratch_types=[
        pltpu.VMEM((NB, CH), jnp.int32),
        pltpu.VMEM((NB, CH), jnp.int32),
        pltpu.VMEM((CH, D), jnp.float32),
        pltpu.VMEM((CH, D), jnp.float32),
        pltpu.VMEM((CH, D), jnp.float32),
        pltpu.VMEM((CH, D), jnp.float32),
        pltpu.VMEM_SHARED((NPAD, D), jnp.float32),
        pltpu.SemaphoreType.DMA,
        pltpu.SemaphoreType.DMA,
        pltpu.SemaphoreType.DMA,
        pltpu.SemaphoreType.DMA,
        pltpu.SemaphoreType.DMA,
        pltpu.SemaphoreType.DMA,
        pltpu.SemaphoreType.DMA,
        pltpu.SemaphoreType.DMA,
    ],
)
def _scatter_kernel(src_hbm, dst_hbm, xs_hbm, s_out,
                    src_blk, dst_blk, r0, r1, r2, r3, agg_sp,
                    g0, g1, g2, g3, s0, s1, s2, s3):
    c = lax.axis_index("c")
    s = lax.axis_index("s")
    wid = c * NS + s
    rows = (r0, r1, r2, r3)
    gsem = (g0, g1, g2, g3)
    ssem = (s0, s1, s2, s3)

    def zr(r, _):
        for j in range(D // L):
            r0[r, pl.ds(j * L, L)] = jnp.zeros((L,), jnp.float32)
        return 0
    lax.fori_loop(0, CH, zr, 0)
    for k in range(SLICE // CH):
        pltpu.sync_copy(r0, agg_sp.at[pl.ds(s * SLICE + k * CH, CH)])
    plsc.subcore_barrier()

    def gstart(i, k):
        pltpu.async_copy(xs_hbm.at[src_blk.at[i]], rows[k], gsem[k])

    def gwait(k):
        pltpu.make_async_copy(xs_hbm.at[src_blk.at[0]], rows[k],
                              gsem[k]).wait()

    def sstart(i, k):
        pltpu.async_copy(rows[k], agg_sp.at[dst_blk.at[i]], ssem[k],
                         add=True)

    def swait(k):
        pltpu.make_async_copy(rows[k], agg_sp.at[dst_blk.at[0]],
                              ssem[k]).wait()

    for blk in range(NBLK):
        pltpu.sync_copy(src_hbm.at[wid, blk], src_blk)
        pltpu.sync_copy(dst_hbm.at[wid, blk], dst_blk)
        for k in range(NBUF):
            gstart(k, k)

        def quad(j, _):
            for k in range(NBUF):
                i = NBUF * j + k
                gwait(k)
                sstart(i, k)
                swait(k)
                gstart(i + NBUF, k)
            return 0
        lax.fori_loop(0, (NB - NBUF) // NBUF, quad, 0)

        for k in range(NBUF):
            gwait(k)
            sstart(NB - NBUF + k, k)
        for k in range(NBUF):
            swait(k)

    plsc.subcore_barrier()
    pltpu.sync_copy(agg_sp.at[pl.ds(s * SLICE, SLICE)],
                    s_out.at[c, pl.ds(s * SLICE, SLICE)])


def _prescale_body(deg_ref, x_ref, adj_ref, xs_ref, dis_ref):
    deg = deg_ref[0, :N] + deg_ref[1, :N] + 1.0 + adj_ref[0, 0]
    dis = lax.rsqrt(deg)
    dis_ref[...] = dis
    xs_ref[...] = x_ref[...] * dis


def _prescale(deg2, x, adj):
    return pl.pallas_call(
        _prescale_body,
        out_shape=[
            jax.ShapeDtypeStruct((N, D), jnp.float32),
            jax.ShapeDtypeStruct((N, 1), jnp.float32),
        ],
    )(deg2, x, adj)


def _combine_body(s_ref, xs_ref, dis_ref, wt_ref, out_ref):
    agg = s_ref[0] + s_ref[1] + xs_ref[...]
    a = agg * dis_ref[...]
    out_ref[...] = jnp.dot(a, wt_ref[...], preferred_element_type=jnp.float32)


def _combine(s2, xs, dis, wt):
    rb = 5000
    grid = N // rb
    return pl.pallas_call(
        _combine_body,
        grid=(grid,),
        in_specs=[
            pl.BlockSpec((NC, rb, D), lambda i: (0, i, 0)),
            pl.BlockSpec((rb, D), lambda i: (i, 0)),
            pl.BlockSpec((rb, 1), lambda i: (i, 0)),
            pl.BlockSpec((D, D), lambda i: (0, 0)),
        ],
        out_specs=pl.BlockSpec((rb, D), lambda i: (i, 0)),
        out_shape=jax.ShapeDtypeStruct((N, D), jnp.float32),
    )(s2, xs, dis, wt)


def kernel(x, edge_index, num_nodes, W):
    pad = jnp.arange(EP - E, dtype=jnp.int32)
    srcp = jnp.concatenate([edge_index[0].astype(jnp.int32), pad % N])
    dstp = jnp.concatenate(
        [edge_index[1].astype(jnp.int32), N + pad % (NPAD - N)])
    src4 = srcp.reshape(NW, NBLK, NB, CH)
    dst4 = dstp.reshape(NW, NBLK, NB, CH)
    adj = (jnp.asarray(num_nodes, jnp.float32) - x.shape[0]).reshape(1, 1)

    deg_p = _degree_kernel(dst4)
    xs, dis = _prescale(deg_p.reshape(NC, NPAD, 1), x, adj)
    s_p = _scatter_kernel(src4, dst4, xs)
    return _combine(s_p, xs, dis, W.T)
